# Initial kernel scaffold; baseline (speedup 1.0000x reference)
import jax
import jax.numpy as jnp
from jax.experimental import pallas as pl


def kernel(x, edge_src, edge_dst, edge_val, label, idx_mask, h0, W, V, alpha, W_out, b_out, u):
    raise NotImplementedError("write your pallas kernel here")


# speedup vs baseline: 3.0663x; 3.0663x over previous
import functools

import jax
import jax.numpy as jnp
from jax import lax
from jax.experimental import pallas as pl
from jax.experimental.pallas import tpu as pltpu
from jax.experimental.pallas import tpu_sc as plsc

N = 10000
D = 128
HOP = 2
DEV = 4
E = 80000
ET = DEV * E
B = 5000

NC = 2
NS = 16
NW = NC * NS
EPW = ET // NW
CHUNK = 80
NCHUNK = EPW // CHUNK

ROWS_PER_TILE = 624
TAIL0 = NS * ROWS_PER_TILE
TAILN = N - TAIL0

SBLK = 128
BP = 5120
RPW = BP // NW
GCH = 80


def _sc_mesh():
    return plsc.VectorSubcoreMesh(core_axis_name="c", subcore_axis_name="s",
                                  num_cores=NC, num_subcores=NS)


def _sc_spmm_body(h_hbm, dst_hbm, src_hbm, val_hbm, z_hbm, out_hbm,
                  acc_sh, dstb, srcb, valb, val_sm, rows, sem):
    c = lax.axis_index("c")
    s = lax.axis_index("s")
    wid = c * NS + s

    row0 = pl.multiple_of(s * ROWS_PER_TILE, 8)
    pltpu.sync_copy(z_hbm, acc_sh.at[pl.ds(row0, ROWS_PER_TILE)])

    @pl.when(s == NS - 1)
    def _zero_tail():
        pltpu.sync_copy(z_hbm.at[pl.ds(0, TAILN)],
                        acc_sh.at[pl.ds(TAIL0, TAILN)])

    plsc.subcore_barrier()

    base = wid * EPW

    def chunk_body(i, carry):
        eb = pl.multiple_of(base + i * CHUNK, 8)
        pltpu.sync_copy(dst_hbm.at[pl.ds(eb, CHUNK)], dstb)
        pltpu.sync_copy(src_hbm.at[pl.ds(eb, CHUNK)], srcb)
        pltpu.sync_copy(val_hbm.at[pl.ds(eb, CHUNK)], valb)
        pltpu.async_copy(h_hbm.at[dstb], rows, sem).wait()
        for eg in range(CHUNK // 16):
            vv = valb[pl.ds(eg * 16, 16)]
            for el in range(16):
                e = eg * 16 + el
                v16 = jnp.full((16,), vv[el], jnp.float32)
                for g in range(D // 16):
                    sl = pl.ds(g * 16, 16)
                    rows[e, sl] = rows[e, sl] * v16
        pltpu.sync_copy(rows, acc_sh.at[srcb], add=True)
        return carry

    lax.fori_loop(0, NCHUNK, chunk_body, 0)
    plsc.subcore_barrier()

    pltpu.sync_copy(acc_sh.at[pl.ds(row0, ROWS_PER_TILE)],
                    out_hbm.at[c].at[pl.ds(row0, ROWS_PER_TILE)])

    @pl.when(s == NS - 1)
    def _copy_tail():
        pltpu.sync_copy(acc_sh.at[pl.ds(TAIL0, TAILN)],
                        out_hbm.at[c].at[pl.ds(TAIL0, TAILN)])


@functools.cache
def _sc_spmm_kernel():
    return pl.kernel(
        _sc_spmm_body,
        out_type=jax.ShapeDtypeStruct((NC, N, D), jnp.float32),
        mesh=_sc_mesh(),
        scratch_types=[
            pltpu.VMEM_SHARED((N, D), jnp.float32),
            pltpu.VMEM((CHUNK,), jnp.int32),
            pltpu.VMEM((CHUNK,), jnp.int32),
            pltpu.VMEM((CHUNK,), jnp.float32),
            pltpu.SMEM((CHUNK,), jnp.float32),
            pltpu.VMEM((CHUNK, D), jnp.float32),
            pltpu.SemaphoreType.DMA,
        ],
    )


def _sc_spmm(h, dst, src, vals, zrows):
    return _sc_spmm_kernel()(h, dst, src, vals, zrows)


def _sc_gather_body(tab_hbm, idx_hbm, out_hbm, idxb, rowsb, sem):
    c = lax.axis_index("c")
    s = lax.axis_index("s")
    wid = c * NS + s
    for j in range(RPW // GCH):
        off = pl.multiple_of(wid * RPW + j * GCH, 8)
        pltpu.sync_copy(idx_hbm.at[pl.ds(off, GCH)], idxb)
        pltpu.async_copy(tab_hbm.at[idxb], rowsb, sem).wait()
        pltpu.sync_copy(rowsb, out_hbm.at[pl.ds(off, GCH)])


@functools.cache
def _sc_gather_kernel():
    return pl.kernel(
        _sc_gather_body,
        out_type=jax.ShapeDtypeStruct((BP, SBLK), jnp.float32),
        mesh=_sc_mesh(),
        scratch_types=[
            pltpu.VMEM((GCH,), jnp.int32),
            pltpu.VMEM((GCH, SBLK), jnp.float32),
            pltpu.SemaphoreType.DMA,
        ],
    )


def _sc_gather(tab, idxp):
    return _sc_gather_kernel()(tab, idxp)


def _vals_body(alpha_ref, ev_ref, out_ref):
    i = pl.program_id(0)
    a = alpha_ref[pl.ds(i, 1), :]
    m = jnp.max(a, axis=1, keepdims=True)
    ex = jnp.exp(a - m)
    coef = ex / jnp.sum(ex, axis=1, keepdims=True)
    scaled = coef.reshape(DEV, 1) * ev_ref[...]
    out_ref[...] = scaled.reshape(1, 1, ET)


def _prep_vals(alpha2, edge_val):
    out = pl.pallas_call(
        _vals_body,
        grid=(HOP,),
        in_specs=[
            pl.BlockSpec((HOP, DEV), lambda i: (0, 0)),
            pl.BlockSpec((DEV, E), lambda i: (0, 0)),
        ],
        out_specs=pl.BlockSpec((1, 1, ET), lambda i: (i, 0, 0)),
        out_shape=jax.ShapeDtypeStruct((HOP, 1, ET), jnp.float32),
    )(alpha2, edge_val)
    return out.reshape(HOP, ET)


RBLK = 2000


def _hop1_body(x_ref, w_ref, acc_ref, v_ref, out_ref):
    h1 = jnp.dot(x_ref[...], w_ref[...], preferred_element_type=jnp.float32)
    a = acc_ref[0] + acc_ref[1]
    h2 = jnp.dot(a, v_ref[...], preferred_element_type=jnp.float32)
    out_ref[...] = jax.nn.sigmoid(h1 + h2)


def _tc_hop1(x, w, acc2, v):
    return pl.pallas_call(
        _hop1_body,
        grid=(N // RBLK,),
        in_specs=[
            pl.BlockSpec((RBLK, D), lambda i: (i, 0)),
            pl.BlockSpec((D, D), lambda i: (0, 0)),
            pl.BlockSpec((NC, RBLK, D), lambda i: (0, i, 0)),
            pl.BlockSpec((D, D), lambda i: (0, 0)),
        ],
        out_specs=pl.BlockSpec((RBLK, D), lambda i: (i, 0)),
        out_shape=jax.ShapeDtypeStruct((N, D), jnp.float32),
    )(x, w, acc2, v)


def _hop2_body(x_ref, w_ref, acc_ref, v_ref, lab_ref, wc_ref, bc_ref,
               sc_ref):
    h1 = jnp.dot(x_ref[...], w_ref[...], preferred_element_type=jnp.float32)
    a = acc_ref[0] + acc_ref[1]
    h2 = jnp.dot(a, v_ref[...], preferred_element_type=jnp.float32)
    h = jax.nn.sigmoid(h1 + h2)
    sc = jnp.dot(h, wc_ref[...], preferred_element_type=jnp.float32)
    sc = sc + bc_ref[...]
    col = lax.broadcasted_iota(jnp.int32, (RBLK, SBLK), 1)
    sc_ref[...] = jnp.where(col == 3, lab_ref[...], sc)


def _tc_hop2(x, w, acc2, v, label, wcomb, bcomb):
    return pl.pallas_call(
        _hop2_body,
        grid=(N // RBLK,),
        in_specs=[
            pl.BlockSpec((RBLK, D), lambda i: (i, 0)),
            pl.BlockSpec((D, D), lambda i: (0, 0)),
            pl.BlockSpec((NC, RBLK, D), lambda i: (0, i, 0)),
            pl.BlockSpec((D, D), lambda i: (0, 0)),
            pl.BlockSpec((RBLK, 1), lambda i: (i, 0)),
            pl.BlockSpec((D, SBLK), lambda i: (0, 0)),
            pl.BlockSpec((1, SBLK), lambda i: (0, 0)),
        ],
        out_specs=pl.BlockSpec((RBLK, SBLK), lambda i: (i, 0)),
        out_shape=jax.ShapeDtypeStruct((N, SBLK), jnp.float32),
    )(x, w, acc2, v, label, wcomb, bcomb)


def _final_body(g_ref, loss_ref, acc_ref):
    g = g_ref[...]
    s = g[:, 0:1]
    l0 = g[:, 1:2]
    l1 = g[:, 2:3]
    m = g[:, 3:4]
    loss_ref[...] = (-jnp.sum(jnp.log(jax.nn.sigmoid(m * s)))).reshape(1, 1)
    pred1 = l1 > l0
    tgt1 = m > 0.0
    acc_ref[...] = jnp.mean((pred1 == tgt1).astype(jnp.float32)).reshape(1, 1)


def _tc_final(g):
    return pl.pallas_call(
        _final_body,
        grid=(1,),
        in_specs=[pl.BlockSpec((B, SBLK), lambda i: (0, 0))],
        out_specs=[
            pl.BlockSpec((1, 1), lambda i: (0, 0)),
            pl.BlockSpec((1, 1), lambda i: (0, 0)),
        ],
        out_shape=[
            jax.ShapeDtypeStruct((1, 1), jnp.float32),
            jax.ShapeDtypeStruct((1, 1), jnp.float32),
        ],
    )(g)


def kernel(x, edge_src, edge_dst, edge_val, label, idx_mask, h0, W, V, alpha,
           W_out, b_out, u):
    src = edge_src.reshape(ET)
    dst = edge_dst.reshape(ET)
    alpha2 = alpha.reshape(HOP, DEV)
    vals2 = _prep_vals(alpha2, edge_val)

    zrows = jnp.zeros((ROWS_PER_TILE, D), jnp.float32)

    wcomb = jnp.concatenate(
        [u.T, W_out, jnp.zeros((D, SBLK - 3), jnp.float32)], axis=1)
    bcomb = jnp.concatenate(
        [jnp.zeros((1, 1), jnp.float32), b_out,
         jnp.zeros((1, SBLK - 3), jnp.float32)], axis=1)

    h = h0
    scores = None
    for i in range(HOP):
        acc2 = _sc_spmm(h, dst, src, vals2[i], zrows)
        if i == 0:
            h = _tc_hop1(x, W[0], acc2, V[0])
        else:
            scores = _tc_hop2(x, W[1], acc2, V[1], label, wcomb, bcomb)

    idxp = jnp.concatenate(
        [idx_mask, jnp.zeros((BP - B,), jnp.int32)])
    g = _sc_gather(scores, idxp)
    loss, acc = _tc_final(g[:B])
    return (loss[0, 0], acc[0, 0])


# speedup vs baseline: 4.2278x; 1.3788x over previous
import functools

import jax
import jax.numpy as jnp
from jax import lax
from jax.experimental import pallas as pl
from jax.experimental.pallas import tpu as pltpu
from jax.experimental.pallas import tpu_sc as plsc

N = 10000
D = 128
HOP = 2
DEV = 4
E = 80000
ET = DEV * E
B = 5000

NC = 2
NS = 16
NW = NC * NS
EPW = ET // NW
CHUNK = 80
NCHUNK = EPW // CHUNK

ROWS_PER_TILE = 624
TAIL0 = NS * ROWS_PER_TILE
TAILN = N - TAIL0

SBLK = 128
BP = 5120
RPW = BP // NW
GCH = 80


def _sc_mesh():
    return plsc.VectorSubcoreMesh(core_axis_name="c", subcore_axis_name="s",
                                  num_cores=NC, num_subcores=NS)


def _sc_spmm_body(h_hbm, dst_hbm, src_hbm, val_hbm, z_hbm, out_hbm,
                  acc_sh, dstA, srcA, valA, rowsA, dstB, srcB, valB, rowsB,
                  semA, semB):
    c = lax.axis_index("c")
    s = lax.axis_index("s")
    wid = c * NS + s

    row0 = pl.multiple_of(s * ROWS_PER_TILE, 8)
    pltpu.sync_copy(z_hbm, acc_sh.at[pl.ds(row0, ROWS_PER_TILE)])

    @pl.when(s == NS - 1)
    def _zero_tail():
        pltpu.sync_copy(z_hbm.at[pl.ds(0, TAILN)],
                        acc_sh.at[pl.ds(TAIL0, TAILN)])

    plsc.subcore_barrier()

    base = wid * EPW

    def copy_idx(i, dstb, srcb, valb):
        eb = pl.multiple_of(base + i * CHUNK, 8)
        pltpu.sync_copy(dst_hbm.at[pl.ds(eb, CHUNK)], dstb)
        pltpu.sync_copy(src_hbm.at[pl.ds(eb, CHUNK)], srcb)
        pltpu.sync_copy(val_hbm.at[pl.ds(eb, CHUNK)], valb)

    def half(i, dstb, srcb, valb, rows, sem):
        @pl.when(i < NCHUNK)
        def _process():
            pltpu.make_async_copy(h_hbm.at[dstb], rows, sem).wait()
            for eg in range(CHUNK // 16):
                vv = valb[pl.ds(eg * 16, 16)]
                for el in range(16):
                    e = eg * 16 + el
                    v16 = jnp.full((16,), vv[el], jnp.float32)
                    for g in range(D // 16):
                        sl = pl.ds(g * 16, 16)
                        rows[e, sl] = rows[e, sl] * v16
            pltpu.sync_copy(rows, acc_sh.at[srcb], add=True)

            @pl.when(i + 2 < NCHUNK)
            def _prefetch():
                copy_idx(i + 2, dstb, srcb, valb)
                pltpu.async_copy(h_hbm.at[dstb], rows, sem)

    copy_idx(0, dstA, srcA, valA)
    pltpu.async_copy(h_hbm.at[dstA], rowsA, semA)
    copy_idx(1, dstB, srcB, valB)
    pltpu.async_copy(h_hbm.at[dstB], rowsB, semB)

    def pair_body(j, carry):
        half(2 * j, dstA, srcA, valA, rowsA, semA)
        half(2 * j + 1, dstB, srcB, valB, rowsB, semB)
        return carry

    lax.fori_loop(0, (NCHUNK + 1) // 2, pair_body, 0)
    plsc.subcore_barrier()

    pltpu.sync_copy(acc_sh.at[pl.ds(row0, ROWS_PER_TILE)],
                    out_hbm.at[c].at[pl.ds(row0, ROWS_PER_TILE)])

    @pl.when(s == NS - 1)
    def _copy_tail():
        pltpu.sync_copy(acc_sh.at[pl.ds(TAIL0, TAILN)],
                        out_hbm.at[c].at[pl.ds(TAIL0, TAILN)])


@functools.cache
def _sc_spmm_kernel():
    return pl.kernel(
        _sc_spmm_body,
        out_type=jax.ShapeDtypeStruct((NC, N, D), jnp.float32),
        mesh=_sc_mesh(),
        scratch_types=[
            pltpu.VMEM_SHARED((N, D), jnp.float32),
            pltpu.VMEM((CHUNK,), jnp.int32),
            pltpu.VMEM((CHUNK,), jnp.int32),
            pltpu.VMEM((CHUNK,), jnp.float32),
            pltpu.VMEM((CHUNK, D), jnp.float32),
            pltpu.VMEM((CHUNK,), jnp.int32),
            pltpu.VMEM((CHUNK,), jnp.int32),
            pltpu.VMEM((CHUNK,), jnp.float32),
            pltpu.VMEM((CHUNK, D), jnp.float32),
            pltpu.SemaphoreType.DMA,
            pltpu.SemaphoreType.DMA,
        ],
    )


def _sc_spmm(h, dst, src, vals, zrows):
    return _sc_spmm_kernel()(h, dst, src, vals, zrows)


def _sc_gather_body(tab_hbm, idx_hbm, out_hbm, idxb, rowsb, sem):
    c = lax.axis_index("c")
    s = lax.axis_index("s")
    wid = c * NS + s
    for j in range(RPW // GCH):
        off = pl.multiple_of(wid * RPW + j * GCH, 8)
        pltpu.sync_copy(idx_hbm.at[pl.ds(off, GCH)], idxb)
        pltpu.async_copy(tab_hbm.at[idxb], rowsb, sem).wait()
        pltpu.sync_copy(rowsb, out_hbm.at[pl.ds(off, GCH)])


@functools.cache
def _sc_gather_kernel():
    return pl.kernel(
        _sc_gather_body,
        out_type=jax.ShapeDtypeStruct((BP, SBLK), jnp.float32),
        mesh=_sc_mesh(),
        scratch_types=[
            pltpu.VMEM((GCH,), jnp.int32),
            pltpu.VMEM((GCH, SBLK), jnp.float32),
            pltpu.SemaphoreType.DMA,
        ],
    )


def _sc_gather(tab, idxp):
    return _sc_gather_kernel()(tab, idxp)


def _vals_body(alpha_ref, ev_ref, out_ref):
    i = pl.program_id(0)
    a = alpha_ref[pl.ds(i, 1), :]
    m = jnp.max(a, axis=1, keepdims=True)
    ex = jnp.exp(a - m)
    coef = ex / jnp.sum(ex, axis=1, keepdims=True)
    scaled = coef.reshape(DEV, 1) * ev_ref[...]
    out_ref[...] = scaled.reshape(1, 1, ET)


def _prep_vals(alpha2, edge_val):
    out = pl.pallas_call(
        _vals_body,
        grid=(HOP,),
        in_specs=[
            pl.BlockSpec((HOP, DEV), lambda i: (0, 0)),
            pl.BlockSpec((DEV, E), lambda i: (0, 0)),
        ],
        out_specs=pl.BlockSpec((1, 1, ET), lambda i: (i, 0, 0)),
        out_shape=jax.ShapeDtypeStruct((HOP, 1, ET), jnp.float32),
    )(alpha2, edge_val)
    return out.reshape(HOP, ET)


RBLK = 2000


def _hop1_body(x_ref, w_ref, acc_ref, v_ref, out_ref):
    h1 = jnp.dot(x_ref[...], w_ref[...], preferred_element_type=jnp.float32)
    a = acc_ref[0] + acc_ref[1]
    h2 = jnp.dot(a, v_ref[...], preferred_element_type=jnp.float32)
    out_ref[...] = jax.nn.sigmoid(h1 + h2)


def _tc_hop1(x, w, acc2, v):
    return pl.pallas_call(
        _hop1_body,
        grid=(N // RBLK,),
        in_specs=[
            pl.BlockSpec((RBLK, D), lambda i: (i, 0)),
            pl.BlockSpec((D, D), lambda i: (0, 0)),
            pl.BlockSpec((NC, RBLK, D), lambda i: (0, i, 0)),
            pl.BlockSpec((D, D), lambda i: (0, 0)),
        ],
        out_specs=pl.BlockSpec((RBLK, D), lambda i: (i, 0)),
        out_shape=jax.ShapeDtypeStruct((N, D), jnp.float32),
    )(x, w, acc2, v)


def _hop2_body(x_ref, w_ref, acc_ref, v_ref, lab_ref, wc_ref, bc_ref,
               sc_ref):
    h1 = jnp.dot(x_ref[...], w_ref[...], preferred_element_type=jnp.float32)
    a = acc_ref[0] + acc_ref[1]
    h2 = jnp.dot(a, v_ref[...], preferred_element_type=jnp.float32)
    h = jax.nn.sigmoid(h1 + h2)
    sc = jnp.dot(h, wc_ref[...], preferred_element_type=jnp.float32)
    sc = sc + bc_ref[...]
    col = lax.broadcasted_iota(jnp.int32, (RBLK, SBLK), 1)
    sc_ref[...] = jnp.where(col == 3, lab_ref[...], sc)


def _tc_hop2(x, w, acc2, v, label, wcomb, bcomb):
    return pl.pallas_call(
        _hop2_body,
        grid=(N // RBLK,),
        in_specs=[
            pl.BlockSpec((RBLK, D), lambda i: (i, 0)),
            pl.BlockSpec((D, D), lambda i: (0, 0)),
            pl.BlockSpec((NC, RBLK, D), lambda i: (0, i, 0)),
            pl.BlockSpec((D, D), lambda i: (0, 0)),
            pl.BlockSpec((RBLK, 1), lambda i: (i, 0)),
            pl.BlockSpec((D, SBLK), lambda i: (0, 0)),
            pl.BlockSpec((1, SBLK), lambda i: (0, 0)),
        ],
        out_specs=pl.BlockSpec((RBLK, SBLK), lambda i: (i, 0)),
        out_shape=jax.ShapeDtypeStruct((N, SBLK), jnp.float32),
    )(x, w, acc2, v, label, wcomb, bcomb)


def _final_body(g_ref, loss_ref, acc_ref):
    g = g_ref[...]
    s = g[:, 0:1]
    l0 = g[:, 1:2]
    l1 = g[:, 2:3]
    m = g[:, 3:4]
    loss_ref[...] = (-jnp.sum(jnp.log(jax.nn.sigmoid(m * s)))).reshape(1, 1)
    pred1 = l1 > l0
    tgt1 = m > 0.0
    acc_ref[...] = jnp.mean((pred1 == tgt1).astype(jnp.float32)).reshape(1, 1)


def _tc_final(g):
    return pl.pallas_call(
        _final_body,
        grid=(1,),
        in_specs=[pl.BlockSpec((B, SBLK), lambda i: (0, 0))],
        out_specs=[
            pl.BlockSpec((1, 1), lambda i: (0, 0)),
            pl.BlockSpec((1, 1), lambda i: (0, 0)),
        ],
        out_shape=[
            jax.ShapeDtypeStruct((1, 1), jnp.float32),
            jax.ShapeDtypeStruct((1, 1), jnp.float32),
        ],
    )(g)


def kernel(x, edge_src, edge_dst, edge_val, label, idx_mask, h0, W, V, alpha,
           W_out, b_out, u):
    src = edge_src.reshape(ET)
    dst = edge_dst.reshape(ET)
    alpha2 = alpha.reshape(HOP, DEV)
    vals2 = _prep_vals(alpha2, edge_val)

    zrows = jnp.zeros((ROWS_PER_TILE, D), jnp.float32)

    wcomb = jnp.concatenate(
        [u.T, W_out, jnp.zeros((D, SBLK - 3), jnp.float32)], axis=1)
    bcomb = jnp.concatenate(
        [jnp.zeros((1, 1), jnp.float32), b_out,
         jnp.zeros((1, SBLK - 3), jnp.float32)], axis=1)

    h = h0
    scores = None
    for i in range(HOP):
        acc2 = _sc_spmm(h, dst, src, vals2[i], zrows)
        if i == 0:
            h = _tc_hop1(x, W[0], acc2, V[0])
        else:
            scores = _tc_hop2(x, W[1], acc2, V[1], label, wcomb, bcomb)

    idxp = jnp.concatenate(
        [idx_mask, jnp.zeros((BP - B,), jnp.int32)])
    g = _sc_gather(scores, idxp)
    loss, acc = _tc_final(g[:B])
    return (loss[0, 0], acc[0, 0])


# speedup vs baseline: 7.1688x; 1.6956x over previous
import functools

import jax
import jax.numpy as jnp
from jax import lax
from jax.experimental import pallas as pl
from jax.experimental.pallas import tpu as pltpu
from jax.experimental.pallas import tpu_sc as plsc

N = 10000
D = 128
HOP = 2
DEV = 4
E = 80000
ET = DEV * E
B = 5000

NC = 2
NS = 16
NW = NC * NS
EPW = ET // NW
CHUNK = 80
NCHUNK = EPW // CHUNK

ROWS_PER_TILE = 624
TAIL0 = NS * ROWS_PER_TILE
TAILN = N - TAIL0

SBLK = 128
BP = 5120
RPW = BP // NW
GCH = 80


def _sc_mesh():
    return plsc.VectorSubcoreMesh(core_axis_name="c", subcore_axis_name="s",
                                  num_cores=NC, num_subcores=NS)


def _sc_spmm_body(h_hbm, dst_hbm, src_hbm, val_hbm, z_hbm, out_hbm,
                  acc_sh, dstall, srcall, valall, srcA, rowsA, srcB, rowsB,
                  semA, semB):
    c = lax.axis_index("c")
    s = lax.axis_index("s")
    wid = c * NS + s

    row0 = pl.multiple_of(s * ROWS_PER_TILE, 8)
    pltpu.sync_copy(z_hbm, acc_sh.at[pl.ds(row0, ROWS_PER_TILE)])

    @pl.when(s == NS - 1)
    def _zero_tail():
        pltpu.sync_copy(z_hbm.at[pl.ds(0, TAILN)],
                        acc_sh.at[pl.ds(TAIL0, TAILN)])

    plsc.subcore_barrier()

    base = pl.multiple_of(wid * EPW, 8)
    pltpu.sync_copy(dst_hbm.at[pl.ds(base, EPW)], dstall)
    pltpu.sync_copy(src_hbm.at[pl.ds(base, EPW)], srcall)
    pltpu.sync_copy(val_hbm.at[pl.ds(base, EPW)], valall)

    def gather_ref(i):
        return h_hbm.at[dstall.at[pl.ds(pl.multiple_of(i * CHUNK, 8), CHUNK)]]

    def half(i, srcb, rows, sem):
        @pl.when(i < NCHUNK)
        def _process():
            off = pl.multiple_of(i * CHUNK, 8)
            for g in range(CHUNK // 16):
                srcb[pl.ds(g * 16, 16)] = srcall[pl.ds(off + g * 16, 16)]
            pltpu.make_async_copy(gather_ref(i), rows, sem).wait()
            for eg in range(CHUNK // 16):
                vv = valall[pl.ds(off + eg * 16, 16)]
                for el in range(16):
                    e = eg * 16 + el
                    v16 = jnp.full((16,), vv[el], jnp.float32)
                    for g in range(D // 16):
                        sl = pl.ds(g * 16, 16)
                        rows[e, sl] = rows[e, sl] * v16
            pltpu.sync_copy(rows, acc_sh.at[srcb], add=True)

            @pl.when(i + 2 < NCHUNK)
            def _prefetch():
                pltpu.async_copy(gather_ref(i + 2), rows, sem)

    pltpu.async_copy(gather_ref(0), rowsA, semA)
    pltpu.async_copy(gather_ref(1), rowsB, semB)

    def pair_body(j, carry):
        half(2 * j, srcA, rowsA, semA)
        half(2 * j + 1, srcB, rowsB, semB)
        return carry

    lax.fori_loop(0, (NCHUNK + 1) // 2, pair_body, 0)
    plsc.subcore_barrier()

    pltpu.sync_copy(acc_sh.at[pl.ds(row0, ROWS_PER_TILE)],
                    out_hbm.at[c].at[pl.ds(row0, ROWS_PER_TILE)])

    @pl.when(s == NS - 1)
    def _copy_tail():
        pltpu.sync_copy(acc_sh.at[pl.ds(TAIL0, TAILN)],
                        out_hbm.at[c].at[pl.ds(TAIL0, TAILN)])


@functools.cache
def _sc_spmm_kernel():
    return pl.kernel(
        _sc_spmm_body,
        out_type=jax.ShapeDtypeStruct((NC, N, D), jnp.float32),
        mesh=_sc_mesh(),
        scratch_types=[
            pltpu.VMEM_SHARED((N, D), jnp.float32),
            pltpu.VMEM((EPW,), jnp.int32),
            pltpu.VMEM((EPW,), jnp.int32),
            pltpu.VMEM((EPW,), jnp.float32),
            pltpu.VMEM((CHUNK,), jnp.int32),
            pltpu.VMEM((CHUNK, D), jnp.float32),
            pltpu.VMEM((CHUNK,), jnp.int32),
            pltpu.VMEM((CHUNK, D), jnp.float32),
            pltpu.SemaphoreType.DMA,
            pltpu.SemaphoreType.DMA,
        ],
    )


def _sc_spmm(h, dst, src, vals, zrows):
    return _sc_spmm_kernel()(h, dst, src, vals, zrows)


def _sc_gather_body(tab_hbm, idx_hbm, out_hbm, idxb, rowsb, sem):
    c = lax.axis_index("c")
    s = lax.axis_index("s")
    wid = c * NS + s
    for j in range(RPW // GCH):
        off = pl.multiple_of(wid * RPW + j * GCH, 8)
        pltpu.sync_copy(idx_hbm.at[pl.ds(off, GCH)], idxb)
        pltpu.async_copy(tab_hbm.at[idxb], rowsb, sem).wait()
        pltpu.sync_copy(rowsb, out_hbm.at[pl.ds(off, GCH)])


@functools.cache
def _sc_gather_kernel():
    return pl.kernel(
        _sc_gather_body,
        out_type=jax.ShapeDtypeStruct((BP, SBLK), jnp.float32),
        mesh=_sc_mesh(),
        scratch_types=[
            pltpu.VMEM((GCH,), jnp.int32),
            pltpu.VMEM((GCH, SBLK), jnp.float32),
            pltpu.SemaphoreType.DMA,
        ],
    )


def _sc_gather(tab, idxp):
    return _sc_gather_kernel()(tab, idxp)


def _vals_body(alpha_ref, ev_ref, out_ref):
    i = pl.program_id(0)
    a = alpha_ref[pl.ds(i, 1), :]
    m = jnp.max(a, axis=1, keepdims=True)
    ex = jnp.exp(a - m)
    coef = ex / jnp.sum(ex, axis=1, keepdims=True)
    scaled = coef.reshape(DEV, 1) * ev_ref[...]
    out_ref[...] = scaled.reshape(1, 1, ET)


def _prep_vals(alpha2, edge_val):
    out = pl.pallas_call(
        _vals_body,
        grid=(HOP,),
        in_specs=[
            pl.BlockSpec((HOP, DEV), lambda i: (0, 0)),
            pl.BlockSpec((DEV, E), lambda i: (0, 0)),
        ],
        out_specs=pl.BlockSpec((1, 1, ET), lambda i: (i, 0, 0)),
        out_shape=jax.ShapeDtypeStruct((HOP, 1, ET), jnp.float32),
    )(alpha2, edge_val)
    return out.reshape(HOP, ET)


RBLK = 2000


def _hop1_body(x_ref, w_ref, acc_ref, v_ref, out_ref):
    h1 = jnp.dot(x_ref[...], w_ref[...], preferred_element_type=jnp.float32)
    a = acc_ref[0] + acc_ref[1]
    h2 = jnp.dot(a, v_ref[...], preferred_element_type=jnp.float32)
    out_ref[...] = jax.nn.sigmoid(h1 + h2)


def _tc_hop1(x, w, acc2, v):
    return pl.pallas_call(
        _hop1_body,
        grid=(N // RBLK,),
        in_specs=[
            pl.BlockSpec((RBLK, D), lambda i: (i, 0)),
            pl.BlockSpec((D, D), lambda i: (0, 0)),
            pl.BlockSpec((NC, RBLK, D), lambda i: (0, i, 0)),
            pl.BlockSpec((D, D), lambda i: (0, 0)),
        ],
        out_specs=pl.BlockSpec((RBLK, D), lambda i: (i, 0)),
        out_shape=jax.ShapeDtypeStruct((N, D), jnp.float32),
    )(x, w, acc2, v)


def _hop2_body(x_ref, w_ref, acc_ref, v_ref, lab_ref, wc_ref, bc_ref,
               sc_ref):
    h1 = jnp.dot(x_ref[...], w_ref[...], preferred_element_type=jnp.float32)
    a = acc_ref[0] + acc_ref[1]
    h2 = jnp.dot(a, v_ref[...], preferred_element_type=jnp.float32)
    h = jax.nn.sigmoid(h1 + h2)
    sc = jnp.dot(h, wc_ref[...], preferred_element_type=jnp.float32)
    sc = sc + bc_ref[...]
    col = lax.broadcasted_iota(jnp.int32, (RBLK, SBLK), 1)
    sc_ref[...] = jnp.where(col == 3, lab_ref[...], sc)


def _tc_hop2(x, w, acc2, v, label, wcomb, bcomb):
    return pl.pallas_call(
        _hop2_body,
        grid=(N // RBLK,),
        in_specs=[
            pl.BlockSpec((RBLK, D), lambda i: (i, 0)),
            pl.BlockSpec((D, D), lambda i: (0, 0)),
            pl.BlockSpec((NC, RBLK, D), lambda i: (0, i, 0)),
            pl.BlockSpec((D, D), lambda i: (0, 0)),
            pl.BlockSpec((RBLK, 1), lambda i: (i, 0)),
            pl.BlockSpec((D, SBLK), lambda i: (0, 0)),
            pl.BlockSpec((1, SBLK), lambda i: (0, 0)),
        ],
        out_specs=pl.BlockSpec((RBLK, SBLK), lambda i: (i, 0)),
        out_shape=jax.ShapeDtypeStruct((N, SBLK), jnp.float32),
    )(x, w, acc2, v, label, wcomb, bcomb)


def _final_body(g_ref, loss_ref, acc_ref):
    g = g_ref[...]
    s = g[:, 0:1]
    l0 = g[:, 1:2]
    l1 = g[:, 2:3]
    m = g[:, 3:4]
    loss_ref[...] = (-jnp.sum(jnp.log(jax.nn.sigmoid(m * s)))).reshape(1, 1)
    pred1 = l1 > l0
    tgt1 = m > 0.0
    acc_ref[...] = jnp.mean((pred1 == tgt1).astype(jnp.float32)).reshape(1, 1)


def _tc_final(g):
    return pl.pallas_call(
        _final_body,
        grid=(1,),
        in_specs=[pl.BlockSpec((B, SBLK), lambda i: (0, 0))],
        out_specs=[
            pl.BlockSpec((1, 1), lambda i: (0, 0)),
            pl.BlockSpec((1, 1), lambda i: (0, 0)),
        ],
        out_shape=[
            jax.ShapeDtypeStruct((1, 1), jnp.float32),
            jax.ShapeDtypeStruct((1, 1), jnp.float32),
        ],
    )(g)


def kernel(x, edge_src, edge_dst, edge_val, label, idx_mask, h0, W, V, alpha,
           W_out, b_out, u):
    src = edge_src.reshape(ET)
    dst = edge_dst.reshape(ET)
    alpha2 = alpha.reshape(HOP, DEV)
    vals2 = _prep_vals(alpha2, edge_val)

    zrows = jnp.zeros((ROWS_PER_TILE, D), jnp.float32)

    wcomb = jnp.concatenate(
        [u.T, W_out, jnp.zeros((D, SBLK - 3), jnp.float32)], axis=1)
    bcomb = jnp.concatenate(
        [jnp.zeros((1, 1), jnp.float32), b_out,
         jnp.zeros((1, SBLK - 3), jnp.float32)], axis=1)

    h = h0
    scores = None
    for i in range(HOP):
        acc2 = _sc_spmm(h, dst, src, vals2[i], zrows)
        if i == 0:
            h = _tc_hop1(x, W[0], acc2, V[0])
        else:
            scores = _tc_hop2(x, W[1], acc2, V[1], label, wcomb, bcomb)

    idxp = jnp.concatenate(
        [idx_mask, jnp.zeros((BP - B,), jnp.int32)])
    g = _sc_gather(scores, idxp)
    loss, acc = _tc_final(g[:B])
    return (loss[0, 0], acc[0, 0])


# speedup vs baseline: 7.5710x; 1.0561x over previous
import functools

import jax
import jax.numpy as jnp
from jax import lax
from jax.experimental import pallas as pl
from jax.experimental.pallas import tpu as pltpu
from jax.experimental.pallas import tpu_sc as plsc

N = 10000
D = 128
HOP = 2
DEV = 4
E = 80000
ET = DEV * E
B = 5000

NC = 2
NS = 16
NW = NC * NS
EPW = ET // NW
CHUNK = 80
SUPER = 2000
NCHUNK = SUPER // CHUNK
NSUPER = EPW // SUPER

ROWS_PER_TILE = 624
TAIL0 = NS * ROWS_PER_TILE
TAILN = N - TAIL0

SBLK = 128
BP = 5120
RPW = BP // NW
GCH = 80


def _sc_mesh():
    return plsc.VectorSubcoreMesh(core_axis_name="c", subcore_axis_name="s",
                                  num_cores=NC, num_subcores=NS)


def _sc_spmm_body(h_hbm, dst_hbm, src_hbm, val_hbm, z_hbm, out_hbm,
                  acc_sh, dstall, srcall, valall,
                  srcA, rowsgA, rowssA, srcB, rowsgB, rowssB,
                  semgA, semsA, semgB, semsB):
    c = lax.axis_index("c")
    s = lax.axis_index("s")
    wid = c * NS + s

    row0 = pl.multiple_of(s * ROWS_PER_TILE, 8)
    pltpu.sync_copy(z_hbm, acc_sh.at[pl.ds(row0, ROWS_PER_TILE)])

    @pl.when(s == NS - 1)
    def _zero_tail():
        pltpu.sync_copy(z_hbm.at[pl.ds(0, TAILN)],
                        acc_sh.at[pl.ds(TAIL0, TAILN)])

    plsc.subcore_barrier()

    base = pl.multiple_of(wid * EPW, 8)

    def gather_ref(i):
        return h_hbm.at[dstall.at[pl.ds(pl.multiple_of(i * CHUNK, 8), CHUNK)]]

    def half(i, srcb, rows_g, rows_s, sem_g, sem_s):
        @pl.when(i < NCHUNK)
        def _process():
            off = pl.multiple_of(i * CHUNK, 8)

            @pl.when(i >= 2)
            def _drain():
                pltpu.make_async_copy(rows_s, acc_sh.at[srcb], sem_s).wait()

            for g in range(CHUNK // 16):
                srcb[pl.ds(g * 16, 16)] = srcall[pl.ds(off + g * 16, 16)]
            pltpu.make_async_copy(gather_ref(i), rows_g, sem_g).wait()
            for eg in range(CHUNK // 16):
                vv = valall[pl.ds(off + eg * 16, 16)]
                for el in range(16):
                    e = eg * 16 + el
                    v16 = jnp.full((16,), vv[el], jnp.float32)
                    for g in range(D // 16):
                        sl = pl.ds(g * 16, 16)
                        rows_s[e, sl] = rows_g[e, sl] * v16

            @pl.when(i + 2 < NCHUNK)
            def _prefetch():
                pltpu.async_copy(gather_ref(i + 2), rows_g, sem_g)

            pltpu.async_copy(rows_s, acc_sh.at[srcb], sem_s, add=True)

    def super_body(sup, carry):
        soff = pl.multiple_of(base + sup * SUPER, 8)
        pltpu.sync_copy(dst_hbm.at[pl.ds(soff, SUPER)], dstall)
        pltpu.sync_copy(src_hbm.at[pl.ds(soff, SUPER)], srcall)
        pltpu.sync_copy(val_hbm.at[pl.ds(soff, SUPER)], valall)

        pltpu.async_copy(gather_ref(0), rowsgA, semgA)
        pltpu.async_copy(gather_ref(1), rowsgB, semgB)

        def pair_body(j, c2):
            half(2 * j, srcA, rowsgA, rowssA, semgA, semsA)
            half(2 * j + 1, srcB, rowsgB, rowssB, semgB, semsB)
            return c2

        lax.fori_loop(0, (NCHUNK + 1) // 2, pair_body, 0)

        pltpu.make_async_copy(rowssA, acc_sh.at[srcA], semsA).wait()
        pltpu.make_async_copy(rowssB, acc_sh.at[srcB], semsB).wait()
        return carry

    lax.fori_loop(0, NSUPER, super_body, 0)
    plsc.subcore_barrier()

    pltpu.sync_copy(acc_sh.at[pl.ds(row0, ROWS_PER_TILE)],
                    out_hbm.at[c].at[pl.ds(row0, ROWS_PER_TILE)])

    @pl.when(s == NS - 1)
    def _copy_tail():
        pltpu.sync_copy(acc_sh.at[pl.ds(TAIL0, TAILN)],
                        out_hbm.at[c].at[pl.ds(TAIL0, TAILN)])


@functools.cache
def _sc_spmm_kernel():
    return pl.kernel(
        _sc_spmm_body,
        out_type=jax.ShapeDtypeStruct((NC, N, D), jnp.float32),
        mesh=_sc_mesh(),
        scratch_types=[
            pltpu.VMEM_SHARED((N, D), jnp.float32),
            pltpu.VMEM((SUPER,), jnp.int32),
            pltpu.VMEM((SUPER,), jnp.int32),
            pltpu.VMEM((SUPER,), jnp.float32),
            pltpu.VMEM((CHUNK,), jnp.int32),
            pltpu.VMEM((CHUNK, D), jnp.float32),
            pltpu.VMEM((CHUNK, D), jnp.float32),
            pltpu.VMEM((CHUNK,), jnp.int32),
            pltpu.VMEM((CHUNK, D), jnp.float32),
            pltpu.VMEM((CHUNK, D), jnp.float32),
            pltpu.SemaphoreType.DMA,
            pltpu.SemaphoreType.DMA,
            pltpu.SemaphoreType.DMA,
            pltpu.SemaphoreType.DMA,
        ],
    )


def _sc_spmm(h, dst, src, vals, zrows):
    return _sc_spmm_kernel()(h, dst, src, vals, zrows)


def _sc_gather_body(tab_hbm, idx_hbm, out_hbm, idxb, rowsb, sem):
    c = lax.axis_index("c")
    s = lax.axis_index("s")
    wid = c * NS + s
    for j in range(RPW // GCH):
        off = pl.multiple_of(wid * RPW + j * GCH, 8)
        pltpu.sync_copy(idx_hbm.at[pl.ds(off, GCH)], idxb)
        pltpu.async_copy(tab_hbm.at[idxb], rowsb, sem).wait()
        pltpu.sync_copy(rowsb, out_hbm.at[pl.ds(off, GCH)])


@functools.cache
def _sc_gather_kernel():
    return pl.kernel(
        _sc_gather_body,
        out_type=jax.ShapeDtypeStruct((BP, SBLK), jnp.float32),
        mesh=_sc_mesh(),
        scratch_types=[
            pltpu.VMEM((GCH,), jnp.int32),
            pltpu.VMEM((GCH, SBLK), jnp.float32),
            pltpu.SemaphoreType.DMA,
        ],
    )


def _sc_gather(tab, idxp):
    return _sc_gather_kernel()(tab, idxp)


def _vals_body(alpha_ref, ev_ref, out_ref):
    i = pl.program_id(0)
    a = alpha_ref[pl.ds(i, 1), :]
    m = jnp.max(a, axis=1, keepdims=True)
    ex = jnp.exp(a - m)
    coef = ex / jnp.sum(ex, axis=1, keepdims=True)
    scaled = coef.reshape(DEV, 1) * ev_ref[...]
    out_ref[...] = scaled.reshape(1, 1, ET)


def _prep_vals(alpha2, edge_val):
    out = pl.pallas_call(
        _vals_body,
        grid=(HOP,),
        in_specs=[
            pl.BlockSpec((HOP, DEV), lambda i: (0, 0)),
            pl.BlockSpec((DEV, E), lambda i: (0, 0)),
        ],
        out_specs=pl.BlockSpec((1, 1, ET), lambda i: (i, 0, 0)),
        out_shape=jax.ShapeDtypeStruct((HOP, 1, ET), jnp.float32),
    )(alpha2, edge_val)
    return out.reshape(HOP, ET)


RBLK = 2000


def _hop1_body(x_ref, w_ref, acc_ref, v_ref, out_ref):
    h1 = jnp.dot(x_ref[...], w_ref[...], preferred_element_type=jnp.float32)
    a = acc_ref[0] + acc_ref[1]
    h2 = jnp.dot(a, v_ref[...], preferred_element_type=jnp.float32)
    out_ref[...] = jax.nn.sigmoid(h1 + h2)


def _tc_hop1(x, w, acc2, v):
    return pl.pallas_call(
        _hop1_body,
        grid=(N // RBLK,),
        in_specs=[
            pl.BlockSpec((RBLK, D), lambda i: (i, 0)),
            pl.BlockSpec((D, D), lambda i: (0, 0)),
            pl.BlockSpec((NC, RBLK, D), lambda i: (0, i, 0)),
            pl.BlockSpec((D, D), lambda i: (0, 0)),
        ],
        out_specs=pl.BlockSpec((RBLK, D), lambda i: (i, 0)),
        out_shape=jax.ShapeDtypeStruct((N, D), jnp.float32),
    )(x, w, acc2, v)


def _hop2_body(x_ref, w_ref, acc_ref, v_ref, lab_ref, wc_ref, bc_ref,
               sc_ref):
    h1 = jnp.dot(x_ref[...], w_ref[...], preferred_element_type=jnp.float32)
    a = acc_ref[0] + acc_ref[1]
    h2 = jnp.dot(a, v_ref[...], preferred_element_type=jnp.float32)
    h = jax.nn.sigmoid(h1 + h2)
    sc = jnp.dot(h, wc_ref[...], preferred_element_type=jnp.float32)
    sc = sc + bc_ref[...]
    col = lax.broadcasted_iota(jnp.int32, (RBLK, SBLK), 1)
    sc_ref[...] = jnp.where(col == 3, lab_ref[...], sc)


def _tc_hop2(x, w, acc2, v, label, wcomb, bcomb):
    return pl.pallas_call(
        _hop2_body,
        grid=(N // RBLK,),
        in_specs=[
            pl.BlockSpec((RBLK, D), lambda i: (i, 0)),
            pl.BlockSpec((D, D), lambda i: (0, 0)),
            pl.BlockSpec((NC, RBLK, D), lambda i: (0, i, 0)),
            pl.BlockSpec((D, D), lambda i: (0, 0)),
            pl.BlockSpec((RBLK, 1), lambda i: (i, 0)),
            pl.BlockSpec((D, SBLK), lambda i: (0, 0)),
            pl.BlockSpec((1, SBLK), lambda i: (0, 0)),
        ],
        out_specs=pl.BlockSpec((RBLK, SBLK), lambda i: (i, 0)),
        out_shape=jax.ShapeDtypeStruct((N, SBLK), jnp.float32),
    )(x, w, acc2, v, label, wcomb, bcomb)


def _final_body(g_ref, loss_ref, acc_ref):
    g = g_ref[...]
    s = g[:, 0:1]
    l0 = g[:, 1:2]
    l1 = g[:, 2:3]
    m = g[:, 3:4]
    loss_ref[...] = (-jnp.sum(jnp.log(jax.nn.sigmoid(m * s)))).reshape(1, 1)
    pred1 = l1 > l0
    tgt1 = m > 0.0
    acc_ref[...] = jnp.mean((pred1 == tgt1).astype(jnp.float32)).reshape(1, 1)


def _tc_final(g):
    return pl.pallas_call(
        _final_body,
        grid=(1,),
        in_specs=[pl.BlockSpec((B, SBLK), lambda i: (0, 0))],
        out_specs=[
            pl.BlockSpec((1, 1), lambda i: (0, 0)),
            pl.BlockSpec((1, 1), lambda i: (0, 0)),
        ],
        out_shape=[
            jax.ShapeDtypeStruct((1, 1), jnp.float32),
            jax.ShapeDtypeStruct((1, 1), jnp.float32),
        ],
    )(g)


def kernel(x, edge_src, edge_dst, edge_val, label, idx_mask, h0, W, V, alpha,
           W_out, b_out, u):
    src = edge_src.reshape(ET)
    dst = edge_dst.reshape(ET)
    alpha2 = alpha.reshape(HOP, DEV)
    vals2 = _prep_vals(alpha2, edge_val)

    zrows = jnp.zeros((ROWS_PER_TILE, D), jnp.float32)

    wcomb = jnp.concatenate(
        [u.T, W_out, jnp.zeros((D, SBLK - 3), jnp.float32)], axis=1)
    bcomb = jnp.concatenate(
        [jnp.zeros((1, 1), jnp.float32), b_out,
         jnp.zeros((1, SBLK - 3), jnp.float32)], axis=1)

    h = h0
    scores = None
    for i in range(HOP):
        acc2 = _sc_spmm(h, dst, src, vals2[i], zrows)
        if i == 0:
            h = _tc_hop1(x, W[0], acc2, V[0])
        else:
            scores = _tc_hop2(x, W[1], acc2, V[1], label, wcomb, bcomb)

    idxp = jnp.concatenate(
        [idx_mask, jnp.zeros((BP - B,), jnp.int32)])
    g = _sc_gather(scores, idxp)
    loss, acc = _tc_final(g[:B])
    return (loss[0, 0], acc[0, 0])
